# baseline (speedup 1.0000x reference)
import functools

import jax
import jax.numpy as jnp
from jax import lax
from jax.experimental import pallas as pl
from jax.experimental.pallas import tpu as pltpu
from jax.experimental.pallas import tpu_sc as plsc

NN = 10000
NE = 160000
F = 256
H = 512
NCORE = 2
NSUB = 16
CH = 112
NCH = 90
EPT = NCH * CH
NNP = 10008
DUMP_A = 632
DUMP_B = NN - 15 * DUMP_A
ZERO_B = NNP - 15 * DUMP_A
ROWB = 1000

_MESH = plsc.VectorSubcoreMesh(
    core_axis_name="c", subcore_axis_name="s", num_cores=NCORE, num_subcores=NSUB
)


@functools.partial(
    pl.kernel,
    out_type=jax.ShapeDtypeStruct((NCORE * NN, 128), jnp.float32),
    mesh=_MESH,
    scratch_types=[
        pltpu.VMEM_SHARED((NNP, 128), jnp.float32),
        pltpu.VMEM((EPT,), jnp.int32),
        pltpu.VMEM((CH,), jnp.int32),
        pltpu.VMEM((CH, 128), jnp.float32),
    ],
)
def _sc_deg(dst2_hbm, ones_hbm, z128_hbm, deg_hbm, acc, dst_all, dst_b, onesbuf):
    c = lax.axis_index("c")
    s = lax.axis_index("s")

    @pl.when(s < NSUB - 1)
    def _():
        pltpu.sync_copy(z128_hbm, acc.at[pl.ds(s * DUMP_A, DUMP_A)])

    @pl.when(s == NSUB - 1)
    def _():
        pltpu.sync_copy(z128_hbm.at[pl.ds(0, ZERO_B)],
                        acc.at[pl.ds((NSUB - 1) * DUMP_A, ZERO_B)])

    pltpu.sync_copy(ones_hbm, onesbuf)
    pltpu.sync_copy(dst2_hbm.at[s], dst_all)
    plsc.subcore_barrier()

    def body(i, carry):
        b = 2 * i + c
        for j in range(CH // 16):
            dst_b[pl.ds(j * 16, 16)] = dst_all[pl.ds(b * CH + j * 16, 16)]
        pltpu.sync_copy(onesbuf, acc.at[dst_b], add=True)
        return carry

    lax.fori_loop(0, NCH // 2, body, 0)
    plsc.subcore_barrier()

    @pl.when(s < NSUB - 1)
    def _():
        pltpu.sync_copy(acc.at[pl.ds(s * DUMP_A, DUMP_A)],
                        deg_hbm.at[pl.ds(c * NN + s * DUMP_A, DUMP_A)])

    @pl.when(s == NSUB - 1)
    def _():
        pltpu.sync_copy(acc.at[pl.ds((NSUB - 1) * DUMP_A, DUMP_B)],
                        deg_hbm.at[pl.ds(c * NN + (NSUB - 1) * DUMP_A, DUMP_B)])


@functools.partial(
    pl.kernel,
    out_type=jax.ShapeDtypeStruct((NCORE * NN, 128), jnp.float32),
    mesh=_MESH,
    scratch_types=[
        pltpu.VMEM_SHARED((NNP, 128), jnp.float32),
        pltpu.VMEM((EPT,), jnp.int32),
        pltpu.VMEM((EPT,), jnp.int32),
        pltpu.VMEM((CH,), jnp.int32),
        pltpu.VMEM((CH, 128), jnp.float32),
        pltpu.VMEM((CH, 128), jnp.float32),
        pltpu.SemaphoreType.DMA,
        pltpu.SemaphoreType.DMA,
    ],
)
def _sc_agg(src2_hbm, dst2_hbm, g_hbm, z128_hbm, agg_hbm, acc, src_all, dst_all,
            dst_b, rb0, rb1, sem0, sem1):
    c = lax.axis_index("c")
    s = lax.axis_index("s")

    @pl.when(s < NSUB - 1)
    def _():
        pltpu.sync_copy(z128_hbm, acc.at[pl.ds(s * DUMP_A, DUMP_A)])

    @pl.when(s == NSUB - 1)
    def _():
        pltpu.sync_copy(z128_hbm.at[pl.ds(0, ZERO_B)],
                        acc.at[pl.ds((NSUB - 1) * DUMP_A, ZERO_B)])

    pltpu.sync_copy(src2_hbm.at[s], src_all)
    pltpu.sync_copy(dst2_hbm.at[s], dst_all)

    off = c * NN

    def add_off(k, carry):
        sl = pl.ds(k * 16, 16)
        src_all[sl] = src_all[sl] + off
        return carry

    lax.fori_loop(0, EPT // 16, add_off, 0)
    plsc.subcore_barrier()

    def start_gather(i, rb, sem):
        pltpu.async_copy(g_hbm.at[src_all.at[pl.ds(i * CH, CH)]], rb, sem)

    def wait_gather(rb, sem):
        pltpu.make_async_copy(g_hbm.at[src_all.at[pl.ds(0, CH)]], rb, sem).wait()

    start_gather(0, rb0, sem0)

    def scatter_add(i, rb):
        for j in range(CH // 16):
            dst_b[pl.ds(j * 16, 16)] = dst_all[pl.ds(i * CH + j * 16, 16)]
        pltpu.sync_copy(rb, acc.at[dst_b], add=True)

    def body(i, carry):
        start_gather(2 * i + 1, rb1, sem1)
        wait_gather(rb0, sem0)
        scatter_add(2 * i, rb0)
        start_gather(2 * i + 2, rb0, sem0)
        wait_gather(rb1, sem1)
        scatter_add(2 * i + 1, rb1)
        return carry

    lax.fori_loop(0, (NCH - 2) // 2, body, 0)
    start_gather(NCH - 1, rb1, sem1)
    wait_gather(rb0, sem0)
    scatter_add(NCH - 2, rb0)
    wait_gather(rb1, sem1)
    scatter_add(NCH - 1, rb1)

    plsc.subcore_barrier()

    @pl.when(s < NSUB - 1)
    def _():
        pltpu.sync_copy(acc.at[pl.ds(s * DUMP_A, DUMP_A)],
                        agg_hbm.at[pl.ds(c * NN + s * DUMP_A, DUMP_A)])

    @pl.when(s == NSUB - 1)
    def _():
        pltpu.sync_copy(acc.at[pl.ds((NSUB - 1) * DUMP_A, DUMP_B)],
                        agg_hbm.at[pl.ds(c * NN + (NSUB - 1) * DUMP_A, DUMP_B)])


def _prep_body(x_ref, deg_ref, g_ref, d8_ref):
    d = lax.rsqrt(deg_ref[0, :, 0:1] + deg_ref[1, :, 0:1] + 1.0)
    g = x_ref[...] * d
    g_ref[0, :, :] = g[:, :128]
    g_ref[1, :, :] = g[:, 128:]
    d8_ref[...] = jnp.broadcast_to(d, (d.shape[0], 8))


def _tc_prep(x, deg2):
    return pl.pallas_call(
        _prep_body,
        grid=(NN // ROWB,),
        in_specs=[
            pl.BlockSpec((ROWB, F), lambda i: (i, 0)),
            pl.BlockSpec((NCORE, ROWB, 128), lambda i: (0, i, 0)),
        ],
        out_specs=[
            pl.BlockSpec((NCORE, ROWB, 128), lambda i: (0, i, 0)),
            pl.BlockSpec((ROWB, 8), lambda i: (i, 0)),
        ],
        out_shape=[
            jax.ShapeDtypeStruct((NCORE, NN, 128), jnp.float32),
            jax.ShapeDtypeStruct((NN, 8), jnp.float32),
        ],
    )(x, deg2)


def _mid_body(agg_ref, g_ref, d8_ref, w1_ref, b1_ref, w2_ref, g2_ref):
    d = d8_ref[:, 0:1]
    u0 = (agg_ref[0, :, :] + g_ref[0, :, :]) * d
    u1 = (agg_ref[1, :, :] + g_ref[1, :, :]) * d
    u = jnp.concatenate([u0, u1], axis=1)
    h = jnp.dot(u, w1_ref[...], preferred_element_type=jnp.float32) + b1_ref[...]
    h = jnp.maximum(h, 0.0)
    t = jnp.dot(h, w2_ref[...], preferred_element_type=jnp.float32)
    g2 = t * d
    g2_ref[0, :, :] = g2[:, :128]
    g2_ref[1, :, :] = g2[:, 128:]


def _tc_mid(agg1, g1, d8, W1, b1, W2):
    return pl.pallas_call(
        _mid_body,
        grid=(NN // ROWB,),
        in_specs=[
            pl.BlockSpec((NCORE, ROWB, 128), lambda i: (0, i, 0)),
            pl.BlockSpec((NCORE, ROWB, 128), lambda i: (0, i, 0)),
            pl.BlockSpec((ROWB, 8), lambda i: (i, 0)),
            pl.BlockSpec((F, H), lambda i: (0, 0)),
            pl.BlockSpec((1, H), lambda i: (0, 0)),
            pl.BlockSpec((H, F), lambda i: (0, 0)),
        ],
        out_specs=pl.BlockSpec((NCORE, ROWB, 128), lambda i: (0, i, 0)),
        out_shape=jax.ShapeDtypeStruct((NCORE, NN, 128), jnp.float32),
    )(agg1, g1, d8, W1, b1, W2)


def _out_body(agg_ref, g_ref, d8_ref, b2_ref, o_ref):
    d = d8_ref[:, 0:1]
    o0 = (agg_ref[0, :, :] + g_ref[0, :, :]) * d
    o1 = (agg_ref[1, :, :] + g_ref[1, :, :]) * d
    o_ref[...] = jnp.concatenate([o0, o1], axis=1) + b2_ref[...]


def _tc_out(agg2, g2, d8, b2):
    return pl.pallas_call(
        _out_body,
        grid=(NN // ROWB,),
        in_specs=[
            pl.BlockSpec((NCORE, ROWB, 128), lambda i: (0, i, 0)),
            pl.BlockSpec((NCORE, ROWB, 128), lambda i: (0, i, 0)),
            pl.BlockSpec((ROWB, 8), lambda i: (i, 0)),
            pl.BlockSpec((1, F), lambda i: (0, 0)),
        ],
        out_specs=pl.BlockSpec((ROWB, F), lambda i: (i, 0)),
        out_shape=jax.ShapeDtypeStruct((NN, F), jnp.float32),
    )(agg2, g2, d8, b2)


def kernel(x, edge_index, W1, b1, W2, b2):
    ei = edge_index.astype(jnp.int32)
    pad = EPT - NE // NSUB
    src2 = jnp.pad(ei[0].reshape(NSUB, NE // NSUB), ((0, 0), (0, pad)),
                   constant_values=0)
    dst2 = jnp.pad(ei[1].reshape(NSUB, NE // NSUB), ((0, 0), (0, pad)),
                   constant_values=NN)
    ones128 = jnp.ones((CH, 128), jnp.float32)
    z128 = jnp.zeros((DUMP_A, 128), jnp.float32)

    deg2 = _sc_deg(dst2, ones128, z128).reshape(NCORE, NN, 128)
    g1, d8 = _tc_prep(x, deg2)
    agg1 = _sc_agg(src2, dst2, g1.reshape(NCORE * NN, 128), z128)
    g2 = _tc_mid(agg1.reshape(NCORE, NN, 128), g1, d8,
                 W1, b1.reshape(1, H), W2)
    agg2 = _sc_agg(src2, dst2, g2.reshape(NCORE * NN, 128), z128)
    return _tc_out(agg2.reshape(NCORE, NN, 128), g2, d8, b2.reshape(1, F))


# speedup vs baseline: 1.0017x; 1.0017x over previous
import functools

import jax
import jax.numpy as jnp
from jax import lax
from jax.experimental import pallas as pl
from jax.experimental.pallas import tpu as pltpu
from jax.experimental.pallas import tpu_sc as plsc

NN = 10000
NE = 160000
F = 256
H = 512
NCORE = 2
NSUB = 16
CH = 112
NCH = 90
EPT = NCH * CH
NNP = 10016
DUMP_A = 632
DUMP_B = NN - 15 * DUMP_A
ZERO_B = NNP - 15 * DUMP_A
ROWB = 1000

_MESH = plsc.VectorSubcoreMesh(
    core_axis_name="c", subcore_axis_name="s", num_cores=NCORE, num_subcores=NSUB
)


@functools.partial(
    pl.kernel,
    out_type=jax.ShapeDtypeStruct((NCORE * NN, 128), jnp.float32),
    mesh=_MESH,
    scratch_types=[
        pltpu.VMEM_SHARED((NNP, 128), jnp.float32),
        pltpu.VMEM((EPT,), jnp.int32),
        pltpu.VMEM((CH,), jnp.int32),
        pltpu.VMEM((CH, 128), jnp.float32),
    ],
)
def _sc_deg(dst2_hbm, ones_hbm, z128_hbm, deg_hbm, acc, dst_all, dst_b, onesbuf):
    c = lax.axis_index("c")
    s = lax.axis_index("s")

    @pl.when(s < NSUB - 1)
    def _():
        pltpu.sync_copy(z128_hbm, acc.at[pl.ds(s * DUMP_A, DUMP_A)])

    @pl.when(s == NSUB - 1)
    def _():
        pltpu.sync_copy(z128_hbm.at[pl.ds(0, ZERO_B)],
                        acc.at[pl.ds((NSUB - 1) * DUMP_A, ZERO_B)])

    pltpu.sync_copy(ones_hbm, onesbuf)
    pltpu.sync_copy(dst2_hbm.at[s], dst_all)
    plsc.subcore_barrier()

    def body(i, carry):
        b = 2 * i + c
        for j in range(CH // 16):
            dst_b[pl.ds(j * 16, 16)] = dst_all[pl.ds(b * CH + j * 16, 16)]
        pltpu.sync_copy(onesbuf, acc.at[dst_b], add=True)
        return carry

    lax.fori_loop(0, NCH // 2, body, 0)
    plsc.subcore_barrier()

    @pl.when(s < NSUB - 1)
    def _():
        pltpu.sync_copy(acc.at[pl.ds(s * DUMP_A, DUMP_A)],
                        deg_hbm.at[pl.ds(c * NN + s * DUMP_A, DUMP_A)])

    @pl.when(s == NSUB - 1)
    def _():
        pltpu.sync_copy(acc.at[pl.ds((NSUB - 1) * DUMP_A, DUMP_B)],
                        deg_hbm.at[pl.ds(c * NN + (NSUB - 1) * DUMP_A, DUMP_B)])


@functools.partial(
    pl.kernel,
    out_type=jax.ShapeDtypeStruct((NCORE * NN, 128), jnp.float32),
    mesh=_MESH,
    scratch_types=[
        pltpu.VMEM_SHARED((NNP, 128), jnp.float32),
        pltpu.VMEM((EPT,), jnp.int32),
        pltpu.VMEM((EPT,), jnp.int32),
        pltpu.VMEM((CH,), jnp.int32),
        pltpu.VMEM((CH, 128), jnp.float32),
        pltpu.VMEM((CH, 128), jnp.float32),
        pltpu.SemaphoreType.DMA,
        pltpu.SemaphoreType.DMA,
    ],
)
def _sc_agg(src2_hbm, dst2_hbm, g_hbm, z128_hbm, agg_hbm, acc, src_all, dst_all,
            dst_b, rb0, rb1, sem0, sem1):
    c = lax.axis_index("c")
    s = lax.axis_index("s")

    @pl.when(s < NSUB - 1)
    def _():
        pltpu.sync_copy(z128_hbm, acc.at[pl.ds(s * DUMP_A, DUMP_A)])

    @pl.when(s == NSUB - 1)
    def _():
        pltpu.sync_copy(z128_hbm.at[pl.ds(0, ZERO_B)],
                        acc.at[pl.ds((NSUB - 1) * DUMP_A, ZERO_B)])

    pltpu.sync_copy(src2_hbm.at[s], src_all)
    pltpu.sync_copy(dst2_hbm.at[s], dst_all)

    off = c * NN

    def add_off(k, carry):
        sl = pl.ds(k * 16, 16)
        src_all[sl] = src_all[sl] + off
        return carry

    lax.fori_loop(0, EPT // 16, add_off, 0)
    plsc.subcore_barrier()

    def start_gather(i, rb, sem):
        pltpu.async_copy(g_hbm.at[src_all.at[pl.ds(i * CH, CH)]], rb, sem)

    def wait_gather(rb, sem):
        pltpu.make_async_copy(g_hbm.at[src_all.at[pl.ds(0, CH)]], rb, sem).wait()

    start_gather(0, rb0, sem0)

    def scatter_add(i, rb):
        for j in range(CH // 16):
            dst_b[pl.ds(j * 16, 16)] = dst_all[pl.ds(i * CH + j * 16, 16)]
        pltpu.sync_copy(rb, acc.at[dst_b], add=True)

    def body(i, carry):
        start_gather(2 * i + 1, rb1, sem1)
        wait_gather(rb0, sem0)
        scatter_add(2 * i, rb0)
        start_gather(2 * i + 2, rb0, sem0)
        wait_gather(rb1, sem1)
        scatter_add(2 * i + 1, rb1)
        return carry

    lax.fori_loop(0, (NCH - 2) // 2, body, 0)
    start_gather(NCH - 1, rb1, sem1)
    wait_gather(rb0, sem0)
    scatter_add(NCH - 2, rb0)
    wait_gather(rb1, sem1)
    scatter_add(NCH - 1, rb1)

    plsc.subcore_barrier()

    @pl.when(s < NSUB - 1)
    def _():
        pltpu.sync_copy(acc.at[pl.ds(s * DUMP_A, DUMP_A)],
                        agg_hbm.at[pl.ds(c * NN + s * DUMP_A, DUMP_A)])

    @pl.when(s == NSUB - 1)
    def _():
        pltpu.sync_copy(acc.at[pl.ds((NSUB - 1) * DUMP_A, DUMP_B)],
                        agg_hbm.at[pl.ds(c * NN + (NSUB - 1) * DUMP_A, DUMP_B)])


def _prep_body(x_ref, deg_ref, g_ref, d8_ref):
    d = lax.rsqrt(deg_ref[0, :, 0:1] + deg_ref[1, :, 0:1] + 1.0)
    g = x_ref[...] * d
    g_ref[0, :, :] = g[:, :128]
    g_ref[1, :, :] = g[:, 128:]
    d8_ref[...] = jnp.broadcast_to(d, (d.shape[0], 8))


def _tc_prep(x, deg2):
    return pl.pallas_call(
        _prep_body,
        grid=(NN // ROWB,),
        in_specs=[
            pl.BlockSpec((ROWB, F), lambda i: (i, 0)),
            pl.BlockSpec((NCORE, ROWB, 128), lambda i: (0, i, 0)),
        ],
        out_specs=[
            pl.BlockSpec((NCORE, ROWB, 128), lambda i: (0, i, 0)),
            pl.BlockSpec((ROWB, 8), lambda i: (i, 0)),
        ],
        out_shape=[
            jax.ShapeDtypeStruct((NCORE, NN, 128), jnp.float32),
            jax.ShapeDtypeStruct((NN, 8), jnp.float32),
        ],
    )(x, deg2)


def _mid_body(agg_ref, g_ref, d8_ref, w1_ref, b1_ref, w2_ref, g2_ref):
    d = d8_ref[:, 0:1]
    u0 = (agg_ref[0, :, :] + g_ref[0, :, :]) * d
    u1 = (agg_ref[1, :, :] + g_ref[1, :, :]) * d
    u = jnp.concatenate([u0, u1], axis=1)
    h = jnp.dot(u, w1_ref[...], preferred_element_type=jnp.float32) + b1_ref[...]
    h = jnp.maximum(h, 0.0)
    t = jnp.dot(h, w2_ref[...], preferred_element_type=jnp.float32)
    g2 = t * d
    g2_ref[0, :, :] = g2[:, :128]
    g2_ref[1, :, :] = g2[:, 128:]


def _tc_mid(agg1, g1, d8, W1, b1, W2):
    return pl.pallas_call(
        _mid_body,
        grid=(NN // ROWB,),
        in_specs=[
            pl.BlockSpec((NCORE, ROWB, 128), lambda i: (0, i, 0)),
            pl.BlockSpec((NCORE, ROWB, 128), lambda i: (0, i, 0)),
            pl.BlockSpec((ROWB, 8), lambda i: (i, 0)),
            pl.BlockSpec((F, H), lambda i: (0, 0)),
            pl.BlockSpec((1, H), lambda i: (0, 0)),
            pl.BlockSpec((H, F), lambda i: (0, 0)),
        ],
        out_specs=pl.BlockSpec((NCORE, ROWB, 128), lambda i: (0, i, 0)),
        out_shape=jax.ShapeDtypeStruct((NCORE, NN, 128), jnp.float32),
    )(agg1, g1, d8, W1, b1, W2)


def _out_body(agg_ref, g_ref, d8_ref, b2_ref, o_ref):
    d = d8_ref[:, 0:1]
    o0 = (agg_ref[0, :, :] + g_ref[0, :, :]) * d
    o1 = (agg_ref[1, :, :] + g_ref[1, :, :]) * d
    o_ref[...] = jnp.concatenate([o0, o1], axis=1) + b2_ref[...]


def _tc_out(agg2, g2, d8, b2):
    return pl.pallas_call(
        _out_body,
        grid=(NN // ROWB,),
        in_specs=[
            pl.BlockSpec((NCORE, ROWB, 128), lambda i: (0, i, 0)),
            pl.BlockSpec((NCORE, ROWB, 128), lambda i: (0, i, 0)),
            pl.BlockSpec((ROWB, 8), lambda i: (i, 0)),
            pl.BlockSpec((1, F), lambda i: (0, 0)),
        ],
        out_specs=pl.BlockSpec((ROWB, F), lambda i: (i, 0)),
        out_shape=jax.ShapeDtypeStruct((NN, F), jnp.float32),
    )(agg2, g2, d8, b2)


def kernel(x, edge_index, W1, b1, W2, b2):
    ei = edge_index.astype(jnp.int32)
    pad = EPT - NE // NSUB
    src2 = jnp.pad(ei[0].reshape(NSUB, NE // NSUB), ((0, 0), (0, pad)),
                   constant_values=0)
    sinks = NN + jnp.arange(NSUB, dtype=jnp.int32)[:, None]
    dst2 = jnp.concatenate(
        [ei[1].reshape(NSUB, NE // NSUB),
         jnp.broadcast_to(sinks, (NSUB, pad))], axis=1)
    ones128 = jnp.ones((CH, 128), jnp.float32)
    z128 = jnp.zeros((DUMP_A, 128), jnp.float32)

    deg2 = _sc_deg(dst2, ones128, z128).reshape(NCORE, NN, 128)
    g1, d8 = _tc_prep(x, deg2)
    agg1 = _sc_agg(src2, dst2, g1.reshape(NCORE * NN, 128), z128)
    g2 = _tc_mid(agg1.reshape(NCORE, NN, 128), g1, d8,
                 W1, b1.reshape(1, H), W2)
    agg2 = _sc_agg(src2, dst2, g2.reshape(NCORE * NN, 128), z128)
    return _tc_out(agg2.reshape(NCORE, NN, 128), g2, d8, b2.reshape(1, F))


# speedup vs baseline: 1.2168x; 1.2147x over previous
import functools

import jax
import jax.numpy as jnp
from jax import lax
from jax.experimental import pallas as pl
from jax.experimental.pallas import tpu as pltpu
from jax.experimental.pallas import tpu_sc as plsc

NN = 10000
NE = 160000
F = 256
H = 512
NCORE = 2
NSUB = 16
CH = 80
NCH = 125
DUMP_A = 632
DUMP_B = NN - 15 * DUMP_A
ROWB = 2000

_MESH = plsc.VectorSubcoreMesh(
    core_axis_name="c", subcore_axis_name="s", num_cores=NCORE, num_subcores=NSUB
)


@functools.partial(
    pl.kernel,
    out_type=jax.ShapeDtypeStruct((NCORE * NN, 128), jnp.float32),
    mesh=_MESH,
    scratch_types=[
        pltpu.VMEM_SHARED((NN, 128), jnp.float32),
        pltpu.VMEM((NCH * CH,), jnp.int32),
        pltpu.VMEM((CH,), jnp.int32),
        pltpu.VMEM((CH, 128), jnp.float32),
    ],
)
def _sc_deg(dst2_hbm, ones_hbm, z128_hbm, deg_hbm, acc, dst_all, dst_b, onesbuf):
    c = lax.axis_index("c")
    s = lax.axis_index("s")

    @pl.when(s < NSUB - 1)
    def _():
        pltpu.sync_copy(z128_hbm, acc.at[pl.ds(s * DUMP_A, DUMP_A)])

    @pl.when(s == NSUB - 1)
    def _():
        pltpu.sync_copy(z128_hbm.at[pl.ds(0, DUMP_B)],
                        acc.at[pl.ds((NSUB - 1) * DUMP_A, DUMP_B)])

    pltpu.sync_copy(ones_hbm, onesbuf)
    pltpu.sync_copy(dst2_hbm.at[s], dst_all)
    plsc.subcore_barrier()

    def body(i, carry):
        b = 2 * i + c

        @pl.when(b < NCH)
        def _():
            for j in range(CH // 16):
                dst_b[pl.ds(j * 16, 16)] = dst_all[pl.ds(b * CH + j * 16, 16)]
            pltpu.sync_copy(onesbuf, acc.at[dst_b], add=True)

        return carry

    lax.fori_loop(0, (NCH + 1) // 2, body, 0)
    plsc.subcore_barrier()

    @pl.when(s < NSUB - 1)
    def _():
        pltpu.sync_copy(acc.at[pl.ds(s * DUMP_A, DUMP_A)],
                        deg_hbm.at[pl.ds(c * NN + s * DUMP_A, DUMP_A)])

    @pl.when(s == NSUB - 1)
    def _():
        pltpu.sync_copy(acc.at[pl.ds((NSUB - 1) * DUMP_A, DUMP_B)],
                        deg_hbm.at[pl.ds(c * NN + (NSUB - 1) * DUMP_A, DUMP_B)])


@functools.partial(
    pl.kernel,
    out_type=jax.ShapeDtypeStruct((NCORE * NN, 128), jnp.float32),
    mesh=_MESH,
    scratch_types=[
        pltpu.VMEM_SHARED((NN, 128), jnp.float32),
        pltpu.VMEM((NCH * CH,), jnp.int32),
        pltpu.VMEM((NCH * CH,), jnp.int32),
        pltpu.VMEM((CH,), jnp.int32),
        pltpu.VMEM((CH, 128), jnp.float32),
        pltpu.VMEM((CH, 128), jnp.float32),
        pltpu.SemaphoreType.DMA,
        pltpu.SemaphoreType.DMA,
    ],
)
def _sc_agg(src2_hbm, dst2_hbm, g_hbm, z128_hbm, agg_hbm, acc, src_all, dst_all,
            dst_b, rb0, rb1, sem0, sem1):
    c = lax.axis_index("c")
    s = lax.axis_index("s")

    @pl.when(s < NSUB - 1)
    def _():
        pltpu.sync_copy(z128_hbm, acc.at[pl.ds(s * DUMP_A, DUMP_A)])

    @pl.when(s == NSUB - 1)
    def _():
        pltpu.sync_copy(z128_hbm.at[pl.ds(0, DUMP_B)],
                        acc.at[pl.ds((NSUB - 1) * DUMP_A, DUMP_B)])

    pltpu.sync_copy(src2_hbm.at[s], src_all)
    pltpu.sync_copy(dst2_hbm.at[s], dst_all)

    off = c * NN

    def add_off(k, carry):
        sl = pl.ds(k * 16, 16)
        src_all[sl] = src_all[sl] + off
        return carry

    lax.fori_loop(0, NCH * CH // 16, add_off, 0)
    plsc.subcore_barrier()

    def start_gather(i, rb, sem):
        pltpu.async_copy(g_hbm.at[src_all.at[pl.ds(i * CH, CH)]], rb, sem)

    def wait_gather(rb, sem):
        pltpu.make_async_copy(g_hbm.at[src_all.at[pl.ds(0, CH)]], rb, sem).wait()

    start_gather(0, rb0, sem0)

    def scatter_add(i, rb):
        for j in range(CH // 16):
            dst_b[pl.ds(j * 16, 16)] = dst_all[pl.ds(i * CH + j * 16, 16)]
        pltpu.sync_copy(rb, acc.at[dst_b], add=True)

    def body(i, carry):
        start_gather(2 * i + 1, rb1, sem1)
        wait_gather(rb0, sem0)
        scatter_add(2 * i, rb0)
        start_gather(2 * i + 2, rb0, sem0)
        wait_gather(rb1, sem1)
        scatter_add(2 * i + 1, rb1)
        return carry

    lax.fori_loop(0, (NCH - 1) // 2, body, 0)
    wait_gather(rb0, sem0)
    scatter_add(NCH - 1, rb0)

    plsc.subcore_barrier()

    @pl.when(s < NSUB - 1)
    def _():
        pltpu.sync_copy(acc.at[pl.ds(s * DUMP_A, DUMP_A)],
                        agg_hbm.at[pl.ds(c * NN + s * DUMP_A, DUMP_A)])

    @pl.when(s == NSUB - 1)
    def _():
        pltpu.sync_copy(acc.at[pl.ds((NSUB - 1) * DUMP_A, DUMP_B)],
                        agg_hbm.at[pl.ds(c * NN + (NSUB - 1) * DUMP_A, DUMP_B)])


def _prep_body(x_ref, deg_ref, g_ref):
    d = lax.rsqrt(deg_ref[0, :, 0:1] + deg_ref[1, :, 0:1] + 1.0)
    g = x_ref[...] * d
    g_ref[0, :, :] = g[:, :128]
    g_ref[1, :, :] = g[:, 128:]


def _tc_prep(x, deg2):
    return pl.pallas_call(
        _prep_body,
        grid=(NN // ROWB,),
        in_specs=[
            pl.BlockSpec((ROWB, F), lambda i: (i, 0)),
            pl.BlockSpec((NCORE, ROWB, 128), lambda i: (0, i, 0)),
        ],
        out_specs=pl.BlockSpec((NCORE, ROWB, 128), lambda i: (0, i, 0)),
        out_shape=jax.ShapeDtypeStruct((NCORE, NN, 128), jnp.float32),
    )(x, deg2)


def _mid_body(agg_ref, g_ref, deg_ref, w1_ref, b1_ref, w2_ref, g2_ref):
    d = lax.rsqrt(deg_ref[0, :, 0:1] + deg_ref[1, :, 0:1] + 1.0)
    u0 = (agg_ref[0, :, :] + g_ref[0, :, :]) * d
    u1 = (agg_ref[1, :, :] + g_ref[1, :, :]) * d
    u = jnp.concatenate([u0, u1], axis=1)
    h = jnp.dot(u, w1_ref[...], preferred_element_type=jnp.float32) + b1_ref[...]
    h = jnp.maximum(h, 0.0)
    t = jnp.dot(h, w2_ref[...], preferred_element_type=jnp.float32)
    g2 = t * d
    g2_ref[0, :, :] = g2[:, :128]
    g2_ref[1, :, :] = g2[:, 128:]


def _tc_mid(agg1, g1, deg2, W1, b1, W2):
    return pl.pallas_call(
        _mid_body,
        grid=(NN // ROWB,),
        in_specs=[
            pl.BlockSpec((NCORE, ROWB, 128), lambda i: (0, i, 0)),
            pl.BlockSpec((NCORE, ROWB, 128), lambda i: (0, i, 0)),
            pl.BlockSpec((NCORE, ROWB, 128), lambda i: (0, i, 0)),
            pl.BlockSpec((F, H), lambda i: (0, 0)),
            pl.BlockSpec((1, H), lambda i: (0, 0)),
            pl.BlockSpec((H, F), lambda i: (0, 0)),
        ],
        out_specs=pl.BlockSpec((NCORE, ROWB, 128), lambda i: (0, i, 0)),
        out_shape=jax.ShapeDtypeStruct((NCORE, NN, 128), jnp.float32),
    )(agg1, g1, deg2, W1, b1, W2)


def _out_body(agg_ref, g_ref, deg_ref, b2_ref, o_ref):
    d = lax.rsqrt(deg_ref[0, :, 0:1] + deg_ref[1, :, 0:1] + 1.0)
    o0 = (agg_ref[0, :, :] + g_ref[0, :, :]) * d
    o1 = (agg_ref[1, :, :] + g_ref[1, :, :]) * d
    o_ref[...] = jnp.concatenate([o0, o1], axis=1) + b2_ref[...]


def _tc_out(agg2, g2, deg2, b2):
    return pl.pallas_call(
        _out_body,
        grid=(NN // ROWB,),
        in_specs=[
            pl.BlockSpec((NCORE, ROWB, 128), lambda i: (0, i, 0)),
            pl.BlockSpec((NCORE, ROWB, 128), lambda i: (0, i, 0)),
            pl.BlockSpec((NCORE, ROWB, 128), lambda i: (0, i, 0)),
            pl.BlockSpec((1, F), lambda i: (0, 0)),
        ],
        out_specs=pl.BlockSpec((ROWB, F), lambda i: (i, 0)),
        out_shape=jax.ShapeDtypeStruct((NN, F), jnp.float32),
    )(agg2, g2, deg2, b2)


def kernel(x, edge_index, W1, b1, W2, b2):
    ei = edge_index.astype(jnp.int32)
    src2 = ei[0].reshape(NSUB, NCH * CH)
    dst2 = ei[1].reshape(NSUB, NCH * CH)
    ones128 = jnp.ones((CH, 128), jnp.float32)
    z128 = jnp.zeros((DUMP_A, 128), jnp.float32)

    deg2 = _sc_deg(dst2, ones128, z128).reshape(NCORE, NN, 128)
    g1 = _tc_prep(x, deg2)
    agg1 = _sc_agg(src2, dst2, g1.reshape(NCORE * NN, 128), z128)
    g2 = _tc_mid(agg1.reshape(NCORE, NN, 128), g1, deg2,
                 W1, b1.reshape(1, H), W2)
    agg2 = _sc_agg(src2, dst2, g2.reshape(NCORE * NN, 128), z128)
    return _tc_out(agg2.reshape(NCORE, NN, 128), g2, deg2, b2.reshape(1, F))


# speedup vs baseline: 1.2221x; 1.0044x over previous
import functools

import jax
import jax.numpy as jnp
from jax import lax
from jax.experimental import pallas as pl
from jax.experimental.pallas import tpu as pltpu
from jax.experimental.pallas import tpu_sc as plsc

NN = 10000
NE = 160000
F = 256
H = 512
NCORE = 2
NSUB = 16
CH = 80
NCH = 125
DUMP_A = 632
DUMP_B = NN - 15 * DUMP_A
ROWB = 2000

_MESH = plsc.VectorSubcoreMesh(
    core_axis_name="c", subcore_axis_name="s", num_cores=NCORE, num_subcores=NSUB
)


@functools.partial(
    pl.kernel,
    out_type=jax.ShapeDtypeStruct((NCORE * NN, 128), jnp.float32),
    mesh=_MESH,
    scratch_types=[
        pltpu.VMEM_SHARED((NN, 128), jnp.float32),
        pltpu.VMEM((NCH * CH,), jnp.int32),
        pltpu.VMEM((CH,), jnp.int32),
        pltpu.VMEM((CH,), jnp.int32),
        pltpu.VMEM((CH, 128), jnp.float32),
        pltpu.SemaphoreType.DMA,
        pltpu.SemaphoreType.DMA,
    ],
)
def _sc_deg(dst2_hbm, ones_hbm, z128_hbm, deg_hbm, acc, dst_all, dst_b0, dst_b1,
            onesbuf, ssem0, ssem1):
    c = lax.axis_index("c")
    s = lax.axis_index("s")

    @pl.when(s < NSUB - 1)
    def _():
        pltpu.sync_copy(z128_hbm, acc.at[pl.ds(s * DUMP_A, DUMP_A)])

    @pl.when(s == NSUB - 1)
    def _():
        pltpu.sync_copy(z128_hbm.at[pl.ds(0, DUMP_B)],
                        acc.at[pl.ds((NSUB - 1) * DUMP_A, DUMP_B)])

    pltpu.sync_copy(ones_hbm, onesbuf)
    pltpu.sync_copy(dst2_hbm.at[s], dst_all)
    plsc.subcore_barrier()

    def prep(b, dstb):
        for j in range(CH // 16):
            dstb[pl.ds(j * 16, 16)] = dst_all[pl.ds(b * CH + j * 16, 16)]

    def issue(dstb, sem):
        pltpu.async_copy(onesbuf, acc.at[dstb], sem, add=True)

    def drain(dstb, sem):
        pltpu.make_async_copy(onesbuf, acc.at[dstb], sem).wait()

    prep(c, dst_b0)
    issue(dst_b0, ssem0)
    prep(2 + c, dst_b1)
    issue(dst_b1, ssem1)

    def body(j, carry):
        drain(dst_b0, ssem0)
        prep(4 * j + c, dst_b0)
        issue(dst_b0, ssem0)
        drain(dst_b1, ssem1)
        prep(4 * j + 2 + c, dst_b1)
        issue(dst_b1, ssem1)
        return carry

    lax.fori_loop(1, 31, body, 0)

    @pl.when(c == 0)
    def _():
        drain(dst_b0, ssem0)
        prep(NCH - 1, dst_b0)
        issue(dst_b0, ssem0)

    drain(dst_b0, ssem0)
    drain(dst_b1, ssem1)
    plsc.subcore_barrier()

    @pl.when(s < NSUB - 1)
    def _():
        pltpu.sync_copy(acc.at[pl.ds(s * DUMP_A, DUMP_A)],
                        deg_hbm.at[pl.ds(c * NN + s * DUMP_A, DUMP_A)])

    @pl.when(s == NSUB - 1)
    def _():
        pltpu.sync_copy(acc.at[pl.ds((NSUB - 1) * DUMP_A, DUMP_B)],
                        deg_hbm.at[pl.ds(c * NN + (NSUB - 1) * DUMP_A, DUMP_B)])


@functools.partial(
    pl.kernel,
    out_type=jax.ShapeDtypeStruct((NCORE * NN, 128), jnp.float32),
    mesh=_MESH,
    scratch_types=[
        pltpu.VMEM_SHARED((NN, 128), jnp.float32),
        pltpu.VMEM((NCH * CH,), jnp.int32),
        pltpu.VMEM((NCH * CH,), jnp.int32),
        pltpu.VMEM((CH,), jnp.int32),
        pltpu.VMEM((CH, 128), jnp.float32),
        pltpu.VMEM((CH, 128), jnp.float32),
        pltpu.SemaphoreType.DMA,
        pltpu.SemaphoreType.DMA,
    ],
)
def _sc_agg(src2_hbm, dst2_hbm, g_hbm, z128_hbm, agg_hbm, acc, src_all, dst_all,
            dst_b, rb0, rb1, sem0, sem1):
    c = lax.axis_index("c")
    s = lax.axis_index("s")

    @pl.when(s < NSUB - 1)
    def _():
        pltpu.sync_copy(z128_hbm, acc.at[pl.ds(s * DUMP_A, DUMP_A)])

    @pl.when(s == NSUB - 1)
    def _():
        pltpu.sync_copy(z128_hbm.at[pl.ds(0, DUMP_B)],
                        acc.at[pl.ds((NSUB - 1) * DUMP_A, DUMP_B)])

    pltpu.sync_copy(src2_hbm.at[s], src_all)
    pltpu.sync_copy(dst2_hbm.at[s], dst_all)

    off = c * NN

    def add_off(k, carry):
        sl = pl.ds(k * 16, 16)
        src_all[sl] = src_all[sl] + off
        return carry

    lax.fori_loop(0, NCH * CH // 16, add_off, 0)
    plsc.subcore_barrier()

    def start_gather(i, rb, sem):
        pltpu.async_copy(g_hbm.at[src_all.at[pl.ds(i * CH, CH)]], rb, sem)

    def wait_gather(rb, sem):
        pltpu.make_async_copy(g_hbm.at[src_all.at[pl.ds(0, CH)]], rb, sem).wait()

    start_gather(0, rb0, sem0)

    def scatter_add(i, rb):
        for j in range(CH // 16):
            dst_b[pl.ds(j * 16, 16)] = dst_all[pl.ds(i * CH + j * 16, 16)]
        pltpu.sync_copy(rb, acc.at[dst_b], add=True)

    def body(i, carry):
        start_gather(2 * i + 1, rb1, sem1)
        wait_gather(rb0, sem0)
        scatter_add(2 * i, rb0)
        start_gather(2 * i + 2, rb0, sem0)
        wait_gather(rb1, sem1)
        scatter_add(2 * i + 1, rb1)
        return carry

    lax.fori_loop(0, (NCH - 1) // 2, body, 0)
    wait_gather(rb0, sem0)
    scatter_add(NCH - 1, rb0)

    plsc.subcore_barrier()

    @pl.when(s < NSUB - 1)
    def _():
        pltpu.sync_copy(acc.at[pl.ds(s * DUMP_A, DUMP_A)],
                        agg_hbm.at[pl.ds(c * NN + s * DUMP_A, DUMP_A)])

    @pl.when(s == NSUB - 1)
    def _():
        pltpu.sync_copy(acc.at[pl.ds((NSUB - 1) * DUMP_A, DUMP_B)],
                        agg_hbm.at[pl.ds(c * NN + (NSUB - 1) * DUMP_A, DUMP_B)])


def _prep_body(x_ref, deg_ref, g_ref):
    d = lax.rsqrt(deg_ref[0, :, 0:1] + deg_ref[1, :, 0:1] + 1.0)
    g = x_ref[...] * d
    g_ref[0, :, :] = g[:, :128]
    g_ref[1, :, :] = g[:, 128:]


def _tc_prep(x, deg2):
    return pl.pallas_call(
        _prep_body,
        grid=(NN // ROWB,),
        in_specs=[
            pl.BlockSpec((ROWB, F), lambda i: (i, 0)),
            pl.BlockSpec((NCORE, ROWB, 128), lambda i: (0, i, 0)),
        ],
        out_specs=pl.BlockSpec((NCORE, ROWB, 128), lambda i: (0, i, 0)),
        out_shape=jax.ShapeDtypeStruct((NCORE, NN, 128), jnp.float32),
    )(x, deg2)


def _mid_body(agg_ref, g_ref, deg_ref, w1_ref, b1_ref, w2_ref, g2_ref):
    d = lax.rsqrt(deg_ref[0, :, 0:1] + deg_ref[1, :, 0:1] + 1.0)
    u0 = (agg_ref[0, :, :] + g_ref[0, :, :]) * d
    u1 = (agg_ref[1, :, :] + g_ref[1, :, :]) * d
    u = jnp.concatenate([u0, u1], axis=1)
    h = jnp.dot(u, w1_ref[...], preferred_element_type=jnp.float32) + b1_ref[...]
    h = jnp.maximum(h, 0.0)
    t = jnp.dot(h, w2_ref[...], preferred_element_type=jnp.float32)
    g2 = t * d
    g2_ref[0, :, :] = g2[:, :128]
    g2_ref[1, :, :] = g2[:, 128:]


def _tc_mid(agg1, g1, deg2, W1, b1, W2):
    return pl.pallas_call(
        _mid_body,
        grid=(NN // ROWB,),
        in_specs=[
            pl.BlockSpec((NCORE, ROWB, 128), lambda i: (0, i, 0)),
            pl.BlockSpec((NCORE, ROWB, 128), lambda i: (0, i, 0)),
            pl.BlockSpec((NCORE, ROWB, 128), lambda i: (0, i, 0)),
            pl.BlockSpec((F, H), lambda i: (0, 0)),
            pl.BlockSpec((1, H), lambda i: (0, 0)),
            pl.BlockSpec((H, F), lambda i: (0, 0)),
        ],
        out_specs=pl.BlockSpec((NCORE, ROWB, 128), lambda i: (0, i, 0)),
        out_shape=jax.ShapeDtypeStruct((NCORE, NN, 128), jnp.float32),
    )(agg1, g1, deg2, W1, b1, W2)


def _out_body(agg_ref, g_ref, deg_ref, b2_ref, o_ref):
    d = lax.rsqrt(deg_ref[0, :, 0:1] + deg_ref[1, :, 0:1] + 1.0)
    o0 = (agg_ref[0, :, :] + g_ref[0, :, :]) * d
    o1 = (agg_ref[1, :, :] + g_ref[1, :, :]) * d
    o_ref[...] = jnp.concatenate([o0, o1], axis=1) + b2_ref[...]


def _tc_out(agg2, g2, deg2, b2):
    return pl.pallas_call(
        _out_body,
        grid=(NN // ROWB,),
        in_specs=[
            pl.BlockSpec((NCORE, ROWB, 128), lambda i: (0, i, 0)),
            pl.BlockSpec((NCORE, ROWB, 128), lambda i: (0, i, 0)),
            pl.BlockSpec((NCORE, ROWB, 128), lambda i: (0, i, 0)),
            pl.BlockSpec((1, F), lambda i: (0, 0)),
        ],
        out_specs=pl.BlockSpec((ROWB, F), lambda i: (i, 0)),
        out_shape=jax.ShapeDtypeStruct((NN, F), jnp.float32),
    )(agg2, g2, deg2, b2)


def kernel(x, edge_index, W1, b1, W2, b2):
    ei = edge_index.astype(jnp.int32)
    src2 = ei[0].reshape(NSUB, NCH * CH)
    dst2 = ei[1].reshape(NSUB, NCH * CH)
    ones128 = jnp.ones((CH, 128), jnp.float32)
    z128 = jnp.zeros((DUMP_A, 128), jnp.float32)

    deg2 = _sc_deg(dst2, ones128, z128).reshape(NCORE, NN, 128)
    g1 = _tc_prep(x, deg2)
    agg1 = _sc_agg(src2, dst2, g1.reshape(NCORE * NN, 128), z128)
    g2 = _tc_mid(agg1.reshape(NCORE, NN, 128), g1, deg2,
                 W1, b1.reshape(1, H), W2)
    agg2 = _sc_agg(src2, dst2, g2.reshape(NCORE * NN, 128), z128)
    return _tc_out(agg2.reshape(NCORE, NN, 128), g2, deg2, b2.reshape(1, F))


# speedup vs baseline: 1.2433x; 1.0174x over previous
import functools

import jax
import jax.numpy as jnp
from jax import lax
from jax.experimental import pallas as pl
from jax.experimental.pallas import tpu as pltpu
from jax.experimental.pallas import tpu_sc as plsc

NN = 10000
NE = 160000
F = 256
H = 512
NCORE = 2
NSUB = 16
CH = 80
NCH = 125
DUMP_A = 632
DUMP_B = NN - 15 * DUMP_A
ROWB = 2000

_MESH = plsc.VectorSubcoreMesh(
    core_axis_name="c", subcore_axis_name="s", num_cores=NCORE, num_subcores=NSUB
)


@functools.partial(
    pl.kernel,
    out_type=jax.ShapeDtypeStruct((NCORE * NN, 128), jnp.float32),
    mesh=_MESH,
    scratch_types=[
        pltpu.VMEM_SHARED((NN, 128), jnp.float32),
        pltpu.VMEM((NCH * CH,), jnp.int32),
        pltpu.VMEM((CH,), jnp.int32),
        pltpu.VMEM((CH,), jnp.int32),
        pltpu.VMEM((CH, 128), jnp.float32),
        pltpu.SemaphoreType.DMA,
        pltpu.SemaphoreType.DMA,
    ],
)
def _sc_deg(dst2_hbm, ones_hbm, z128_hbm, deg_hbm, acc, dst_all, dst_b0, dst_b1,
            onesbuf, ssem0, ssem1):
    c = lax.axis_index("c")
    s = lax.axis_index("s")

    @pl.when(s < NSUB - 1)
    def _():
        pltpu.sync_copy(z128_hbm, acc.at[pl.ds(s * DUMP_A, DUMP_A)])

    @pl.when(s == NSUB - 1)
    def _():
        pltpu.sync_copy(z128_hbm.at[pl.ds(0, DUMP_B)],
                        acc.at[pl.ds((NSUB - 1) * DUMP_A, DUMP_B)])

    pltpu.sync_copy(ones_hbm, onesbuf)
    pltpu.sync_copy(dst2_hbm.at[s], dst_all)
    plsc.subcore_barrier()

    def prep(b, dstb):
        for j in range(CH // 16):
            dstb[pl.ds(j * 16, 16)] = dst_all[pl.ds(b * CH + j * 16, 16)]

    def issue(dstb, sem):
        pltpu.async_copy(onesbuf, acc.at[dstb], sem, add=True)

    def drain(dstb, sem):
        pltpu.make_async_copy(onesbuf, acc.at[dstb], sem).wait()

    prep(c, dst_b0)
    issue(dst_b0, ssem0)
    prep(2 + c, dst_b1)
    issue(dst_b1, ssem1)

    def body(j, carry):
        drain(dst_b0, ssem0)
        prep(4 * j + c, dst_b0)
        issue(dst_b0, ssem0)
        drain(dst_b1, ssem1)
        prep(4 * j + 2 + c, dst_b1)
        issue(dst_b1, ssem1)
        return carry

    lax.fori_loop(1, 31, body, 0)

    @pl.when(c == 0)
    def _():
        drain(dst_b0, ssem0)
        prep(NCH - 1, dst_b0)
        issue(dst_b0, ssem0)

    drain(dst_b0, ssem0)
    drain(dst_b1, ssem1)
    plsc.subcore_barrier()

    @pl.when(s < NSUB - 1)
    def _():
        pltpu.sync_copy(acc.at[pl.ds(s * DUMP_A, DUMP_A)],
                        deg_hbm.at[pl.ds(c * NN + s * DUMP_A, DUMP_A)])

    @pl.when(s == NSUB - 1)
    def _():
        pltpu.sync_copy(acc.at[pl.ds((NSUB - 1) * DUMP_A, DUMP_B)],
                        deg_hbm.at[pl.ds(c * NN + (NSUB - 1) * DUMP_A, DUMP_B)])


@functools.partial(
    pl.kernel,
    out_type=jax.ShapeDtypeStruct((NCORE * NN, 128), jnp.float32),
    mesh=_MESH,
    scratch_types=[
        pltpu.VMEM_SHARED((NN, 128), jnp.float32),
        pltpu.VMEM((NCH * CH,), jnp.int32),
        pltpu.VMEM((NCH * CH,), jnp.int32),
        pltpu.VMEM((CH,), jnp.int32),
        pltpu.VMEM((CH, 128), jnp.float32),
        pltpu.VMEM((CH, 128), jnp.float32),
        pltpu.SemaphoreType.DMA,
        pltpu.SemaphoreType.DMA,
    ],
)
def _sc_agg(src2_hbm, dst2_hbm, g_hbm, z128_hbm, agg_hbm, acc, src_all, dst_all,
            dst_b, rb0, rb1, sem0, sem1):
    c = lax.axis_index("c")
    s = lax.axis_index("s")

    @pl.when(s < NSUB - 1)
    def _():
        pltpu.sync_copy(z128_hbm, acc.at[pl.ds(s * DUMP_A, DUMP_A)])

    @pl.when(s == NSUB - 1)
    def _():
        pltpu.sync_copy(z128_hbm.at[pl.ds(0, DUMP_B)],
                        acc.at[pl.ds((NSUB - 1) * DUMP_A, DUMP_B)])

    pltpu.sync_copy(src2_hbm.at[s], src_all)
    pltpu.sync_copy(dst2_hbm.at[s], dst_all)

    off = c * NN

    def add_off(k, carry):
        for u in range(5):
            sl = pl.ds((k * 5 + u) * 16, 16)
            src_all[sl] = src_all[sl] + off
        return carry

    lax.fori_loop(0, NCH * CH // 80, add_off, 0)
    plsc.subcore_barrier()

    def start_gather(i, rb, sem):
        pltpu.async_copy(g_hbm.at[src_all.at[pl.ds(i * CH, CH)]], rb, sem)

    def wait_gather(rb, sem):
        pltpu.make_async_copy(g_hbm.at[src_all.at[pl.ds(0, CH)]], rb, sem).wait()

    start_gather(0, rb0, sem0)

    def prep_idx(i):
        for j in range(CH // 16):
            dst_b[pl.ds(j * 16, 16)] = dst_all[pl.ds(i * CH + j * 16, 16)]

    def body(i, carry):
        start_gather(2 * i + 1, rb1, sem1)
        prep_idx(2 * i)
        wait_gather(rb0, sem0)
        pltpu.sync_copy(rb0, acc.at[dst_b], add=True)
        start_gather(2 * i + 2, rb0, sem0)
        prep_idx(2 * i + 1)
        wait_gather(rb1, sem1)
        pltpu.sync_copy(rb1, acc.at[dst_b], add=True)
        return carry

    lax.fori_loop(0, (NCH - 1) // 2, body, 0)
    prep_idx(NCH - 1)
    wait_gather(rb0, sem0)
    pltpu.sync_copy(rb0, acc.at[dst_b], add=True)

    plsc.subcore_barrier()

    @pl.when(s < NSUB - 1)
    def _():
        pltpu.sync_copy(acc.at[pl.ds(s * DUMP_A, DUMP_A)],
                        agg_hbm.at[pl.ds(c * NN + s * DUMP_A, DUMP_A)])

    @pl.when(s == NSUB - 1)
    def _():
        pltpu.sync_copy(acc.at[pl.ds((NSUB - 1) * DUMP_A, DUMP_B)],
                        agg_hbm.at[pl.ds(c * NN + (NSUB - 1) * DUMP_A, DUMP_B)])


def _prep_body(x_ref, deg_ref, g_ref):
    d = lax.rsqrt(deg_ref[0, :, 0:1] + deg_ref[1, :, 0:1] + 1.0)
    g = x_ref[...] * d
    g_ref[0, :, :] = g[:, :128]
    g_ref[1, :, :] = g[:, 128:]


def _tc_prep(x, deg2):
    return pl.pallas_call(
        _prep_body,
        grid=(NN // ROWB,),
        in_specs=[
            pl.BlockSpec((ROWB, F), lambda i: (i, 0)),
            pl.BlockSpec((NCORE, ROWB, 128), lambda i: (0, i, 0)),
        ],
        out_specs=pl.BlockSpec((NCORE, ROWB, 128), lambda i: (0, i, 0)),
        out_shape=jax.ShapeDtypeStruct((NCORE, NN, 128), jnp.float32),
    )(x, deg2)


def _mid_body(agg_ref, g_ref, deg_ref, w1_ref, b1_ref, w2_ref, g2_ref):
    d = lax.rsqrt(deg_ref[0, :, 0:1] + deg_ref[1, :, 0:1] + 1.0)
    u0 = (agg_ref[0, :, :] + g_ref[0, :, :]) * d
    u1 = (agg_ref[1, :, :] + g_ref[1, :, :]) * d
    u = jnp.concatenate([u0, u1], axis=1)
    h = jnp.dot(u, w1_ref[...], preferred_element_type=jnp.float32) + b1_ref[...]
    h = jnp.maximum(h, 0.0)
    t = jnp.dot(h, w2_ref[...], preferred_element_type=jnp.float32)
    g2 = t * d
    g2_ref[0, :, :] = g2[:, :128]
    g2_ref[1, :, :] = g2[:, 128:]


def _tc_mid(agg1, g1, deg2, W1, b1, W2):
    return pl.pallas_call(
        _mid_body,
        grid=(NN // ROWB,),
        in_specs=[
            pl.BlockSpec((NCORE, ROWB, 128), lambda i: (0, i, 0)),
            pl.BlockSpec((NCORE, ROWB, 128), lambda i: (0, i, 0)),
            pl.BlockSpec((NCORE, ROWB, 128), lambda i: (0, i, 0)),
            pl.BlockSpec((F, H), lambda i: (0, 0)),
            pl.BlockSpec((1, H), lambda i: (0, 0)),
            pl.BlockSpec((H, F), lambda i: (0, 0)),
        ],
        out_specs=pl.BlockSpec((NCORE, ROWB, 128), lambda i: (0, i, 0)),
        out_shape=jax.ShapeDtypeStruct((NCORE, NN, 128), jnp.float32),
    )(agg1, g1, deg2, W1, b1, W2)


def _out_body(agg_ref, g_ref, deg_ref, b2_ref, o_ref):
    d = lax.rsqrt(deg_ref[0, :, 0:1] + deg_ref[1, :, 0:1] + 1.0)
    o0 = (agg_ref[0, :, :] + g_ref[0, :, :]) * d
    o1 = (agg_ref[1, :, :] + g_ref[1, :, :]) * d
    o_ref[...] = jnp.concatenate([o0, o1], axis=1) + b2_ref[...]


def _tc_out(agg2, g2, deg2, b2):
    return pl.pallas_call(
        _out_body,
        grid=(NN // ROWB,),
        in_specs=[
            pl.BlockSpec((NCORE, ROWB, 128), lambda i: (0, i, 0)),
            pl.BlockSpec((NCORE, ROWB, 128), lambda i: (0, i, 0)),
            pl.BlockSpec((NCORE, ROWB, 128), lambda i: (0, i, 0)),
            pl.BlockSpec((1, F), lambda i: (0, 0)),
        ],
        out_specs=pl.BlockSpec((ROWB, F), lambda i: (i, 0)),
        out_shape=jax.ShapeDtypeStruct((NN, F), jnp.float32),
    )(agg2, g2, deg2, b2)


def kernel(x, edge_index, W1, b1, W2, b2):
    ei = edge_index.astype(jnp.int32)
    src2 = ei[0].reshape(NSUB, NCH * CH)
    dst2 = ei[1].reshape(NSUB, NCH * CH)
    ones128 = jnp.ones((CH, 128), jnp.float32)
    z128 = jnp.zeros((DUMP_A, 128), jnp.float32)

    deg2 = _sc_deg(dst2, ones128, z128).reshape(NCORE, NN, 128)
    g1 = _tc_prep(x, deg2)
    agg1 = _sc_agg(src2, dst2, g1.reshape(NCORE * NN, 128), z128)
    g2 = _tc_mid(agg1.reshape(NCORE, NN, 128), g1, deg2,
                 W1, b1.reshape(1, H), W2)
    agg2 = _sc_agg(src2, dst2, g2.reshape(NCORE * NN, 128), z128)
    return _tc_out(agg2.reshape(NCORE, NN, 128), g2, deg2, b2.reshape(1, F))


# speedup vs baseline: 1.2496x; 1.0051x over previous
import functools

import jax
import jax.numpy as jnp
from jax import lax
from jax.experimental import pallas as pl
from jax.experimental.pallas import tpu as pltpu
from jax.experimental.pallas import tpu_sc as plsc

NN = 10000
NE = 160000
F = 256
H = 512
NCORE = 2
NSUB = 16
CH = 80
NCH = 125
DUMP_A = 632
DUMP_B = NN - 15 * DUMP_A
ROWB = 5000

_MESH = plsc.VectorSubcoreMesh(
    core_axis_name="c", subcore_axis_name="s", num_cores=NCORE, num_subcores=NSUB
)


@functools.partial(
    pl.kernel,
    out_type=jax.ShapeDtypeStruct((NCORE * NN, 128), jnp.float32),
    mesh=_MESH,
    scratch_types=[
        pltpu.VMEM_SHARED((NN, 128), jnp.float32),
        pltpu.VMEM((NCH * CH,), jnp.int32),
        pltpu.VMEM((CH,), jnp.int32),
        pltpu.VMEM((CH,), jnp.int32),
        pltpu.VMEM((CH, 128), jnp.float32),
        pltpu.SemaphoreType.DMA,
        pltpu.SemaphoreType.DMA,
    ],
)
def _sc_deg(dst2_hbm, ones_hbm, z128_hbm, deg_hbm, acc, dst_all, dst_b0, dst_b1,
            onesbuf, ssem0, ssem1):
    c = lax.axis_index("c")
    s = lax.axis_index("s")

    @pl.when(s < NSUB - 1)
    def _():
        pltpu.sync_copy(z128_hbm, acc.at[pl.ds(s * DUMP_A, DUMP_A)])

    @pl.when(s == NSUB - 1)
    def _():
        pltpu.sync_copy(z128_hbm.at[pl.ds(0, DUMP_B)],
                        acc.at[pl.ds((NSUB - 1) * DUMP_A, DUMP_B)])

    pltpu.sync_copy(ones_hbm, onesbuf)
    pltpu.sync_copy(dst2_hbm.at[s], dst_all)
    plsc.subcore_barrier()

    def prep(b, dstb):
        for j in range(CH // 16):
            dstb[pl.ds(j * 16, 16)] = dst_all[pl.ds(b * CH + j * 16, 16)]

    def issue(dstb, sem):
        pltpu.async_copy(onesbuf, acc.at[dstb], sem, add=True)

    def drain(dstb, sem):
        pltpu.make_async_copy(onesbuf, acc.at[dstb], sem).wait()

    prep(c, dst_b0)
    issue(dst_b0, ssem0)
    prep(2 + c, dst_b1)
    issue(dst_b1, ssem1)

    def body(j, carry):
        drain(dst_b0, ssem0)
        prep(4 * j + c, dst_b0)
        issue(dst_b0, ssem0)
        drain(dst_b1, ssem1)
        prep(4 * j + 2 + c, dst_b1)
        issue(dst_b1, ssem1)
        return carry

    lax.fori_loop(1, 31, body, 0)

    @pl.when(c == 0)
    def _():
        drain(dst_b0, ssem0)
        prep(NCH - 1, dst_b0)
        issue(dst_b0, ssem0)

    drain(dst_b0, ssem0)
    drain(dst_b1, ssem1)
    plsc.subcore_barrier()

    @pl.when(s < NSUB - 1)
    def _():
        pltpu.sync_copy(acc.at[pl.ds(s * DUMP_A, DUMP_A)],
                        deg_hbm.at[pl.ds(c * NN + s * DUMP_A, DUMP_A)])

    @pl.when(s == NSUB - 1)
    def _():
        pltpu.sync_copy(acc.at[pl.ds((NSUB - 1) * DUMP_A, DUMP_B)],
                        deg_hbm.at[pl.ds(c * NN + (NSUB - 1) * DUMP_A, DUMP_B)])


@functools.partial(
    pl.kernel,
    out_type=jax.ShapeDtypeStruct((NCORE * NN, 128), jnp.float32),
    mesh=_MESH,
    scratch_types=[
        pltpu.VMEM_SHARED((NN, 128), jnp.float32),
        pltpu.VMEM((NCH * CH,), jnp.int32),
        pltpu.VMEM((NCH * CH,), jnp.int32),
        pltpu.VMEM((CH,), jnp.int32),
        pltpu.VMEM((CH, 128), jnp.float32),
        pltpu.VMEM((CH, 128), jnp.float32),
        pltpu.SemaphoreType.DMA,
        pltpu.SemaphoreType.DMA,
    ],
)
def _sc_agg(src2_hbm, dst2_hbm, g_hbm, z128_hbm, agg_hbm, acc, src_all, dst_all,
            dst_b, rb0, rb1, sem0, sem1):
    c = lax.axis_index("c")
    s = lax.axis_index("s")

    @pl.when(s < NSUB - 1)
    def _():
        pltpu.sync_copy(z128_hbm, acc.at[pl.ds(s * DUMP_A, DUMP_A)])

    @pl.when(s == NSUB - 1)
    def _():
        pltpu.sync_copy(z128_hbm.at[pl.ds(0, DUMP_B)],
                        acc.at[pl.ds((NSUB - 1) * DUMP_A, DUMP_B)])

    pltpu.sync_copy(src2_hbm.at[s], src_all)
    pltpu.sync_copy(dst2_hbm.at[s], dst_all)

    off = c * NN

    def add_off(k, carry):
        for u in range(5):
            sl = pl.ds((k * 5 + u) * 16, 16)
            src_all[sl] = src_all[sl] + off
        return carry

    lax.fori_loop(0, NCH * CH // 80, add_off, 0)
    plsc.subcore_barrier()

    def start_gather(i, rb, sem):
        pltpu.async_copy(g_hbm.at[src_all.at[pl.ds(i * CH, CH)]], rb, sem)

    def wait_gather(rb, sem):
        pltpu.make_async_copy(g_hbm.at[src_all.at[pl.ds(0, CH)]], rb, sem).wait()

    start_gather(0, rb0, sem0)

    def prep_idx(i):
        for j in range(CH // 16):
            dst_b[pl.ds(j * 16, 16)] = dst_all[pl.ds(i * CH + j * 16, 16)]

    def body(i, carry):
        start_gather(2 * i + 1, rb1, sem1)
        prep_idx(2 * i)
        wait_gather(rb0, sem0)
        pltpu.sync_copy(rb0, acc.at[dst_b], add=True)
        start_gather(2 * i + 2, rb0, sem0)
        prep_idx(2 * i + 1)
        wait_gather(rb1, sem1)
        pltpu.sync_copy(rb1, acc.at[dst_b], add=True)
        return carry

    lax.fori_loop(0, (NCH - 1) // 2, body, 0)
    prep_idx(NCH - 1)
    wait_gather(rb0, sem0)
    pltpu.sync_copy(rb0, acc.at[dst_b], add=True)

    plsc.subcore_barrier()

    @pl.when(s < NSUB - 1)
    def _():
        pltpu.sync_copy(acc.at[pl.ds(s * DUMP_A, DUMP_A)],
                        agg_hbm.at[pl.ds(c * NN + s * DUMP_A, DUMP_A)])

    @pl.when(s == NSUB - 1)
    def _():
        pltpu.sync_copy(acc.at[pl.ds((NSUB - 1) * DUMP_A, DUMP_B)],
                        agg_hbm.at[pl.ds(c * NN + (NSUB - 1) * DUMP_A, DUMP_B)])


def _prep_body(x_ref, deg_ref, g_ref):
    d = lax.rsqrt(deg_ref[0, :, 0:1] + deg_ref[1, :, 0:1] + 1.0)
    g = x_ref[...] * d
    g_ref[0, :, :] = g[:, :128]
    g_ref[1, :, :] = g[:, 128:]


def _tc_prep(x, deg2):
    return pl.pallas_call(
        _prep_body,
        grid=(NN // ROWB,),
        in_specs=[
            pl.BlockSpec((ROWB, F), lambda i: (i, 0)),
            pl.BlockSpec((NCORE, ROWB, 128), lambda i: (0, i, 0)),
        ],
        out_specs=pl.BlockSpec((NCORE, ROWB, 128), lambda i: (0, i, 0)),
        out_shape=jax.ShapeDtypeStruct((NCORE, NN, 128), jnp.float32),
    )(x, deg2)


def _mid_body(agg_ref, g_ref, deg_ref, w1_ref, b1_ref, w2_ref, g2_ref):
    d = lax.rsqrt(deg_ref[0, :, 0:1] + deg_ref[1, :, 0:1] + 1.0)
    u0 = (agg_ref[0, :, :] + g_ref[0, :, :]) * d
    u1 = (agg_ref[1, :, :] + g_ref[1, :, :]) * d
    u = jnp.concatenate([u0, u1], axis=1)
    h = jnp.dot(u, w1_ref[...], preferred_element_type=jnp.float32) + b1_ref[...]
    h = jnp.maximum(h, 0.0)
    t = jnp.dot(h, w2_ref[...], preferred_element_type=jnp.float32)
    g2 = t * d
    g2_ref[0, :, :] = g2[:, :128]
    g2_ref[1, :, :] = g2[:, 128:]


def _tc_mid(agg1, g1, deg2, W1, b1, W2):
    return pl.pallas_call(
        _mid_body,
        grid=(NN // ROWB,),
        in_specs=[
            pl.BlockSpec((NCORE, ROWB, 128), lambda i: (0, i, 0)),
            pl.BlockSpec((NCORE, ROWB, 128), lambda i: (0, i, 0)),
            pl.BlockSpec((NCORE, ROWB, 128), lambda i: (0, i, 0)),
            pl.BlockSpec((F, H), lambda i: (0, 0)),
            pl.BlockSpec((1, H), lambda i: (0, 0)),
            pl.BlockSpec((H, F), lambda i: (0, 0)),
        ],
        out_specs=pl.BlockSpec((NCORE, ROWB, 128), lambda i: (0, i, 0)),
        out_shape=jax.ShapeDtypeStruct((NCORE, NN, 128), jnp.float32),
    )(agg1, g1, deg2, W1, b1, W2)


def _out_body(agg_ref, g_ref, deg_ref, b2_ref, o_ref):
    d = lax.rsqrt(deg_ref[0, :, 0:1] + deg_ref[1, :, 0:1] + 1.0)
    o0 = (agg_ref[0, :, :] + g_ref[0, :, :]) * d
    o1 = (agg_ref[1, :, :] + g_ref[1, :, :]) * d
    o_ref[...] = jnp.concatenate([o0, o1], axis=1) + b2_ref[...]


def _tc_out(agg2, g2, deg2, b2):
    return pl.pallas_call(
        _out_body,
        grid=(NN // ROWB,),
        in_specs=[
            pl.BlockSpec((NCORE, ROWB, 128), lambda i: (0, i, 0)),
            pl.BlockSpec((NCORE, ROWB, 128), lambda i: (0, i, 0)),
            pl.BlockSpec((NCORE, ROWB, 128), lambda i: (0, i, 0)),
            pl.BlockSpec((1, F), lambda i: (0, 0)),
        ],
        out_specs=pl.BlockSpec((ROWB, F), lambda i: (i, 0)),
        out_shape=jax.ShapeDtypeStruct((NN, F), jnp.float32),
    )(agg2, g2, deg2, b2)


def kernel(x, edge_index, W1, b1, W2, b2):
    ei = edge_index.astype(jnp.int32)
    src2 = ei[0].reshape(NSUB, NCH * CH)
    dst2 = ei[1].reshape(NSUB, NCH * CH)
    ones128 = jnp.ones((CH, 128), jnp.float32)
    z128 = jnp.zeros((DUMP_A, 128), jnp.float32)

    deg2 = _sc_deg(dst2, ones128, z128).reshape(NCORE, NN, 128)
    g1 = _tc_prep(x, deg2)
    agg1 = _sc_agg(src2, dst2, g1.reshape(NCORE * NN, 128), z128)
    g2 = _tc_mid(agg1.reshape(NCORE, NN, 128), g1, deg2,
                 W1, b1.reshape(1, H), W2)
    agg2 = _sc_agg(src2, dst2, g2.reshape(NCORE * NN, 128), z128)
    return _tc_out(agg2.reshape(NCORE, NN, 128), g2, deg2, b2.reshape(1, F))
